# baseline (speedup 1.0000x reference)
import functools

import jax
import jax.numpy as jnp
from jax import lax
from jax.experimental import pallas as pl
from jax.experimental.pallas import tpu as pltpu
from jax.experimental.pallas import tpu_sc as plsc

_T = 256


def _routing_body(ef_ref, dest_ref, ig_ref, eg_ref, lo_ref, hi_ref, ff_ref,
                  par_ref, ega1_ref, ega2_ref, egb1_ref, egb2_ref):
    nrow, nlane = ef_ref.shape
    N = nrow * nlane
    T = _T
    nb = N // T
    G = ig_ref.shape[1]
    E = 8

    ef = ef_ref[...]
    f32 = jnp.float32

    ri = lax.broadcasted_iota(jnp.int32, (nlane, nlane), 0)
    ci = lax.broadcasted_iota(jnp.int32, (nlane, nlane), 1)
    U = (ri <= ci).astype(f32)
    rb = lax.broadcasted_iota(jnp.int32, (nrow, nrow), 0)
    cb = lax.broadcasted_iota(jnp.int32, (nrow, nrow), 1)
    Ls = (rb > cb).astype(f32)

    rank = jnp.zeros((nrow, nlane), f32)
    base = jnp.zeros((nrow, nlane), f32)
    off = jnp.zeros((), f32)
    offs = [off]
    for e in range(E):
        m = (ef == e)
        mf = m.astype(f32)
        intra = jax.lax.dot_general(mf, U, (((1,), (0,)), ((), ())),
                                    preferred_element_type=f32)
        rowtot = jnp.sum(mf, axis=1, keepdims=True)
        carry = jax.lax.dot_general(Ls, rowtot, (((1,), (0,)), ((), ())),
                                    preferred_element_type=f32)
        ce = intra + carry
        rank = rank + jnp.where(m, ce, 0.0)
        base = base + jnp.where(m, off, 0.0)
        off = off + jnp.sum(mf)
        offs.append(off)
    dest_ref[...] = (base + rank - 1.0).astype(jnp.int32)

    bs = lax.broadcasted_iota(jnp.int32, (1, nb), 1).astype(f32) * T
    e_first = jnp.zeros((1, nb), f32)
    e_last = jnp.zeros((1, nb), f32)
    for e in range(1, E + 1):
        e_first = e_first + (offs[e] <= bs).astype(f32)
        e_last = e_last + (offs[e] <= bs + (T - 1)).astype(f32)
    npairs = e_last - e_first + 1.0
    lb = lax.broadcasted_iota(jnp.int32, (1, nb), 1)
    Unb_r = lax.broadcasted_iota(jnp.int32, (nb, nb), 0)
    Unb_c = lax.broadcasted_iota(jnp.int32, (nb, nb), 1)
    Unb = (Unb_r <= Unb_c).astype(f32)
    cs = jax.lax.dot_general(npairs, Unb, (((1,), (0,)), ((), ())),
                             preferred_element_type=f32)
    gstart = cs - npairs
    total = jnp.sum(npairs)

    def pick(vec, i):
        return jnp.sum(jnp.where(lb == i, vec, 0.0))

    g = lax.broadcasted_iota(jnp.int32, (1, G), 1).astype(f32)
    ig = jnp.full((1, G), -1.0, f32)
    ig2 = jnp.full((1, G), -1.0, f32)
    eg = jnp.zeros((1, G), f32)
    el = jnp.zeros((1, G), f32)
    for i in range(nb):
        gs_i = pick(gstart, i)
        ig = ig + (gs_i <= g).astype(f32)
        ig2 = ig2 + (gs_i <= g - 1.0).astype(f32)
    for i in range(nb):
        sel = (ig == i).astype(f32)
        eg = eg + sel * (pick(e_first, i) + g - pick(gstart, i))
        el = el + sel * pick(e_last, i)
    eg = jnp.minimum(eg, el)
    lo = jnp.zeros((1, G), f32)
    hi = jnp.zeros((1, G), f32)
    for e in range(E):
        sel = (eg == e).astype(f32)
        lo = lo + sel * offs[e]
        hi = hi + sel * offs[e + 1]
    lo = jnp.clip(lo, ig * T, (ig + 1.0) * T)
    hi = jnp.clip(hi, ig * T, (ig + 1.0) * T)
    hi = jnp.where(g < total, hi, lo)
    ff = jnp.logical_or(g == 0, ig != ig2)

    Sr = lax.broadcasted_iota(jnp.int32, (G, G), 0)
    Sc = lax.broadcasted_iota(jnp.int32, (G, G), 1)
    Sh = (Sr == Sc - 1).astype(f32)
    UG = (Sr <= Sc).astype(f32)
    eg_prev = jax.lax.dot_general(eg, Sh, (((1,), (0,)), ((), ())),
                                  preferred_element_type=f32)
    ch = jnp.where(jnp.logical_or(g == 0, eg != eg_prev), 1.0, 0.0)
    r = jax.lax.dot_general(ch, UG, (((1,), (0,)), ((), ())),
                            preferred_element_type=f32) - 1.0
    rmax = jnp.sum(ch) - 1.0
    par = r - 2.0 * jnp.floor(r * 0.5)
    rA = jnp.where(par == 0, r, jnp.minimum(r + 1.0, rmax))
    rB = jnp.where(par == 1, r, jnp.minimum(r + 1.0, rmax))
    egA = jnp.zeros((1, G), f32)
    egB = jnp.zeros((1, G), f32)
    for k in range(E):
        er_k = jnp.sum(ch * (r == k).astype(f32) * eg)
        egA = egA + (rA == k).astype(f32) * er_k
        egB = egB + (rB == k).astype(f32) * er_k
    egA2 = jax.lax.dot_general(egA, Sh, (((1,), (0,)), ((), ())),
                               preferred_element_type=f32)
    egB2 = jax.lax.dot_general(egB, Sh, (((1,), (0,)), ((), ())),
                               preferred_element_type=f32)
    egA2 = jnp.where(g == 0, egA, egA2)
    egB2 = jnp.where(g == 0, egB, egB2)

    i32 = jnp.int32
    ig_ref[...] = ig.astype(i32)
    eg_ref[...] = eg.astype(i32)
    lo_ref[...] = lo.astype(i32)
    hi_ref[...] = hi.astype(i32)
    ff_ref[...] = ff.astype(i32)
    par_ref[...] = par.astype(i32)
    ega1_ref[...] = egA.astype(i32)
    ega2_ref[...] = egA2.astype(i32)
    egb1_ref[...] = egB.astype(i32)
    egb2_ref[...] = egB2.astype(i32)


def _routing(ef2, G):
    nrow, nlane = ef2.shape
    i32 = jnp.int32
    return pl.pallas_call(
        _routing_body,
        out_shape=[jax.ShapeDtypeStruct((nrow, nlane), i32)] +
                  [jax.ShapeDtypeStruct((1, G), i32)] * 10,
    )(ef2)


def _gmm_body(ig_r, eg_r, lo_r, hi_r, ff_r, par_r, ega1_r, ega2_r,
              egb1_r, egb2_r,
              xs_r, w1a_r, w1b_r, b1_r, w2a_r, w2b_r, b2_r, ys_r):
    g = pl.program_id(0)
    T = ys_r.shape[0]
    E = b1_r.shape[0]
    lo = lo_r[0, g]
    hi = hi_r[0, g]
    base = ig_r[0, g] * T

    def do(w1_r, w2_r):
        eg = eg_r[0, g]
        rows = jax.lax.broadcasted_iota(jnp.int32, (E, 1), 0)
        b1 = jnp.sum(jnp.where(rows == eg, b1_r[...], 0.0), 0, keepdims=True)
        b2 = jnp.sum(jnp.where(rows == eg, b2_r[...], 0.0), 0, keepdims=True)
        x = xs_r[...]
        h = jax.lax.dot_general(x, w1_r[0], (((1,), (1,)), ((), ())),
                                preferred_element_type=jnp.float32)
        h = jnp.maximum(h + b1, 0.0)
        y = jax.lax.dot_general(h, w2_r[0], (((1,), (1,)), ((), ())),
                                preferred_element_type=jnp.float32)
        y = y + b2
        full = (lo == base) & (hi == base + T)

        @pl.when(full)
        def _():
            ys_r[...] = y

        @pl.when(~full)
        def _():
            r = base + jax.lax.broadcasted_iota(jnp.int32, (T, 1), 0)
            ym = jnp.where((r >= lo) & (r < hi), y, 0.0)

            @pl.when(ff_r[0, g] == 1)
            def _():
                ys_r[...] = ym

            @pl.when(ff_r[0, g] == 0)
            def _():
                ys_r[...] += ym

    p = par_r[0, g]

    @pl.when((lo < hi) & (p == 0))
    def _():
        do(w1a_r, w2a_r)

    @pl.when((lo < hi) & (p == 1))
    def _():
        do(w1b_r, w2b_r)


def _grouped_ffn(xs, W1, b1, W2, b2, sched):
    N, H = xs.shape
    E = W1.shape[0]
    ig = sched[0]
    G = ig.shape[1]
    T = _T
    imap_x = lambda g, *s: (s[0][0, g], 0)
    imap_a1 = lambda g, *s: (s[6][0, g], 0, 0)
    imap_a2 = lambda g, *s: (s[7][0, g], 0, 0)
    imap_b1 = lambda g, *s: (s[8][0, g], 0, 0)
    imap_b2 = lambda g, *s: (s[9][0, g], 0, 0)
    imap_c = lambda g, *s: (0, 0)
    grid_spec = pltpu.PrefetchScalarGridSpec(
        num_scalar_prefetch=10,
        grid=(G,),
        in_specs=[
            pl.BlockSpec((T, H), imap_x),
            pl.BlockSpec((1, H, H), imap_a1),
            pl.BlockSpec((1, H, H), imap_b1),
            pl.BlockSpec((E, H), imap_c),
            pl.BlockSpec((1, H, H), imap_a2),
            pl.BlockSpec((1, H, H), imap_b2),
            pl.BlockSpec((E, H), imap_c),
        ],
        out_specs=pl.BlockSpec((T, H), imap_x),
    )
    return pl.pallas_call(
        _gmm_body,
        grid_spec=grid_spec,
        out_shape=jax.ShapeDtypeStruct((N, H), jnp.float32),
    )(*sched, xs, W1, W1, b1, W2, W2, b2)


def _dg16(v, idx):
    dnums = lax.GatherDimensionNumbers(
        offset_dims=(), collapsed_slice_dims=(0,), start_index_map=(0,))
    return lax.gather(v, idx[:, None], dnums, (1,),
                      mode=lax.GatherScatterMode.PROMISE_IN_BOUNDS)


def _sc_scatter(x2d, dest2, K):
    NT, H = x2d.shape
    NW, P = dest2.shape
    N = NW * P
    tpw = NT // NW

    @functools.partial(
        pl.kernel,
        mesh=plsc.VectorSubcoreMesh(core_axis_name="c", subcore_axis_name="s"),
        out_type=jax.ShapeDtypeStruct((N, H), jnp.float32),
        scratch_types=[pltpu.VMEM((P,), jnp.int32),
                       pltpu.VMEM((tpw,), jnp.int32),
                       pltpu.VMEM((tpw,), jnp.int32),
                       pltpu.VMEM((tpw, H), jnp.float32),
                       pltpu.SemaphoreType.DMA],
    )
    def k(x_hbm, dest_hbm, xs_hbm, dch_v, ev_v, od_v, rows_v, sem):
        wid = lax.axis_index("s") * 2 + lax.axis_index("c")
        dx = pltpu.async_copy(x_hbm.at[pl.ds(wid * tpw, tpw)], rows_v, sem)
        pltpu.sync_copy(dest_hbm.at[wid], dch_v)
        ip = lax.iota(jnp.int32, 16)
        half = (ip < 8)
        evi = (ip % 8) * 2
        for c in range(tpw // 16):
            c0 = dch_v[pl.ds(32 * c, 16)]
            c1 = dch_v[pl.ds(32 * c + 16, 16)]
            ev_v[pl.ds(16 * c, 16)] = jnp.where(
                half, _dg16(c0, evi), _dg16(c1, evi))
            od_v[pl.ds(16 * c, 16)] = jnp.where(
                half, _dg16(c0, evi + 1), _dg16(c1, evi + 1))
        dx.wait()
        d1 = pltpu.async_copy(rows_v, xs_hbm.at[ev_v], sem)
        d2 = pltpu.async_copy(rows_v, xs_hbm.at[od_v], sem)
        d1.wait()
        d2.wait()

    assert K == 2 and P == K * tpw
    return k(x2d, dest2)


def _sc_combine(ys, dest2, wflat, NT):
    N, H = ys.shape
    NW, P = dest2.shape
    tpw = NT // NW
    nq = 2
    hp = P // nq
    tph = tpw // nq

    @functools.partial(
        pl.kernel,
        mesh=plsc.VectorSubcoreMesh(core_axis_name="c", subcore_axis_name="s"),
        out_type=jax.ShapeDtypeStruct((NT, H), jnp.float32),
        scratch_types=[pltpu.VMEM((hp,), jnp.int32),
                       pltpu.VMEM((hp,), jnp.int32),
                       pltpu.VMEM((P,), jnp.float32),
                       pltpu.VMEM((hp, H), jnp.float32),
                       pltpu.VMEM((hp, H), jnp.float32),
                       pltpu.VMEM((tph, H), jnp.float32),
                       pltpu.SemaphoreType.DMA,
                       pltpu.SemaphoreType.DMA],
    )
    def k(ys_hbm, dest_hbm, w_hbm, out_hbm,
          idxA, idxB, w_v, bufA, bufB, obuf_v, semA, semB):
        wid = lax.axis_index("s") * 2 + lax.axis_index("c")
        pltpu.sync_copy(w_hbm.at[pl.ds(wid * P, P)], w_v)
        idxs = [idxA, idxB]
        bufs = [bufA, bufB]
        sems = [semA, semB]
        pltpu.sync_copy(dest_hbm.at[wid, pl.ds(0, hp)], idxA)
        dma = [pltpu.async_copy(ys_hbm.at[idxA], bufA, semA), None]
        for q in range(nq):
            cur = q % 2
            if q + 1 < nq:
                nxt = (q + 1) % 2
                pltpu.sync_copy(
                    dest_hbm.at[wid, pl.ds((q + 1) * hp, hp)], idxs[nxt])
                dma[nxt] = pltpu.async_copy(
                    ys_hbm.at[idxs[nxt]], bufs[nxt], sems[nxt])
            dma[cur].wait()
            buf_v = bufs[cur]

            @plsc.parallel_loop(0, tph, 1, unroll=8)
            def body(j):
                jj = j + q * tph
                b = jnp.minimum(2 * jj, P - 16)
                o = 2 * jj - b
                wv = w_v[pl.ds(b, 16)]
                z = jnp.zeros((16,), jnp.int32)
                s0 = _dg16(wv, z + o)
                s1 = _dg16(wv, z + o + 1)
                for c in range(H // 16):
                    s = pl.ds(c * 16, 16)
                    obuf_v[j, s] = s0 * buf_v[2 * j, s] + s1 * buf_v[2 * j + 1, s]
            pltpu.sync_copy(obuf_v, out_hbm.at[pl.ds(wid * tpw + q * tph, tph)])

    return k(ys, dest2, wflat)


def kernel(hidden_states, top_k_index, top_k_weights, W1, b1, W2, b2):
    B, S, H = hidden_states.shape
    E = W1.shape[0]
    NT = B * S
    K = top_k_index.shape[-1]
    N = NT * K
    NW = 32
    G = N // _T + E - 1

    x2d = hidden_states.reshape(NT, H)
    ef2 = top_k_index.astype(jnp.int32).reshape(N // 128, 128)
    wflat = top_k_weights.reshape(N)

    routed = _routing(ef2, G)
    dest2 = routed[0]
    sched = routed[1:]
    destw = dest2.reshape(NW, N // NW)

    xs = _sc_scatter(x2d, destw, K)
    ys = _grouped_ffn(xs, W1, b1, W2, b2, sched)
    out = _sc_combine(ys, destw, wflat, NT)
    return out.reshape(B, S, H)


# speedup vs baseline: 1.0297x; 1.0297x over previous
import functools

import jax
import jax.numpy as jnp
from jax import lax
from jax.experimental import pallas as pl
from jax.experimental.pallas import tpu as pltpu
from jax.experimental.pallas import tpu_sc as plsc

_T = 256


def _routing_body(ef_ref, dest_ref, ig_ref, eg_ref, lo_ref, hi_ref, ff_ref,
                  par_ref, ega1_ref, ega2_ref, egb1_ref, egb2_ref):
    nrow, nlane = ef_ref.shape
    N = nrow * nlane
    T = _T
    nb = N // T
    G = ig_ref.shape[1]
    E = 8

    ef = ef_ref[...]
    f32 = jnp.float32

    ri = lax.broadcasted_iota(jnp.int32, (nlane, nlane), 0)
    ci = lax.broadcasted_iota(jnp.int32, (nlane, nlane), 1)
    U = (ri <= ci).astype(f32)
    rb = lax.broadcasted_iota(jnp.int32, (nrow, nrow), 0)
    cb = lax.broadcasted_iota(jnp.int32, (nrow, nrow), 1)
    Ls = (rb > cb).astype(f32)

    rank = jnp.zeros((nrow, nlane), f32)
    base = jnp.zeros((nrow, nlane), f32)
    off = jnp.zeros((), f32)
    offs = [off]
    for e in range(E):
        m = (ef == e)
        mf = m.astype(f32)
        intra = jax.lax.dot_general(mf, U, (((1,), (0,)), ((), ())),
                                    preferred_element_type=f32)
        rowtot = jnp.sum(mf, axis=1, keepdims=True)
        carry = jax.lax.dot_general(Ls, rowtot, (((1,), (0,)), ((), ())),
                                    preferred_element_type=f32)
        ce = intra + carry
        rank = rank + jnp.where(m, ce, 0.0)
        base = base + jnp.where(m, off, 0.0)
        off = off + jnp.sum(mf)
        offs.append(off)
    dest_ref[...] = (base + rank - 1.0).astype(jnp.int32)

    bs = lax.broadcasted_iota(jnp.int32, (1, nb), 1).astype(f32) * T
    e_first = jnp.zeros((1, nb), f32)
    e_last = jnp.zeros((1, nb), f32)
    for e in range(1, E + 1):
        e_first = e_first + (offs[e] <= bs).astype(f32)
        e_last = e_last + (offs[e] <= bs + (T - 1)).astype(f32)
    npairs = e_last - e_first + 1.0
    lb = lax.broadcasted_iota(jnp.int32, (1, nb), 1)
    Unb_r = lax.broadcasted_iota(jnp.int32, (nb, nb), 0)
    Unb_c = lax.broadcasted_iota(jnp.int32, (nb, nb), 1)
    Unb = (Unb_r <= Unb_c).astype(f32)
    cs = jax.lax.dot_general(npairs, Unb, (((1,), (0,)), ((), ())),
                             preferred_element_type=f32)
    gstart = cs - npairs
    total = jnp.sum(npairs)

    def pick(vec, i):
        return jnp.sum(jnp.where(lb == i, vec, 0.0))

    g = lax.broadcasted_iota(jnp.int32, (1, G), 1).astype(f32)
    ig = jnp.full((1, G), -1.0, f32)
    ig2 = jnp.full((1, G), -1.0, f32)
    eg = jnp.zeros((1, G), f32)
    el = jnp.zeros((1, G), f32)
    for i in range(nb):
        gs_i = pick(gstart, i)
        ig = ig + (gs_i <= g).astype(f32)
        ig2 = ig2 + (gs_i <= g - 1.0).astype(f32)
    for i in range(nb):
        sel = (ig == i).astype(f32)
        eg = eg + sel * (pick(e_first, i) + g - pick(gstart, i))
        el = el + sel * pick(e_last, i)
    eg = jnp.minimum(eg, el)
    lo = jnp.zeros((1, G), f32)
    hi = jnp.zeros((1, G), f32)
    for e in range(E):
        sel = (eg == e).astype(f32)
        lo = lo + sel * offs[e]
        hi = hi + sel * offs[e + 1]
    lo = jnp.clip(lo, ig * T, (ig + 1.0) * T)
    hi = jnp.clip(hi, ig * T, (ig + 1.0) * T)
    hi = jnp.where(g < total, hi, lo)
    ff = jnp.logical_or(g == 0, ig != ig2)

    Sr = lax.broadcasted_iota(jnp.int32, (G, G), 0)
    Sc = lax.broadcasted_iota(jnp.int32, (G, G), 1)
    Sh = (Sr == Sc - 1).astype(f32)
    UG = (Sr <= Sc).astype(f32)
    eg_prev = jax.lax.dot_general(eg, Sh, (((1,), (0,)), ((), ())),
                                  preferred_element_type=f32)
    ch = jnp.where(jnp.logical_or(g == 0, eg != eg_prev), 1.0, 0.0)
    r = jax.lax.dot_general(ch, UG, (((1,), (0,)), ((), ())),
                            preferred_element_type=f32) - 1.0
    rmax = jnp.sum(ch) - 1.0
    par = r - 2.0 * jnp.floor(r * 0.5)
    rA = jnp.where(par == 0, r, jnp.minimum(r + 1.0, rmax))
    rB = jnp.where(par == 1, r, jnp.minimum(r + 1.0, rmax))
    egA = jnp.zeros((1, G), f32)
    egB = jnp.zeros((1, G), f32)
    for k in range(E):
        er_k = jnp.sum(ch * (r == k).astype(f32) * eg)
        egA = egA + (rA == k).astype(f32) * er_k
        egB = egB + (rB == k).astype(f32) * er_k
    egA2 = jax.lax.dot_general(egA, Sh, (((1,), (0,)), ((), ())),
                               preferred_element_type=f32)
    egB2 = jax.lax.dot_general(egB, Sh, (((1,), (0,)), ((), ())),
                               preferred_element_type=f32)
    egA2 = jnp.where(g == 0, egA, egA2)
    egB2 = jnp.where(g == 0, egB, egB2)

    i32 = jnp.int32
    ig_ref[...] = ig.astype(i32)
    eg_ref[...] = eg.astype(i32)
    lo_ref[...] = lo.astype(i32)
    hi_ref[...] = hi.astype(i32)
    ff_ref[...] = ff.astype(i32)
    par_ref[...] = par.astype(i32)
    ega1_ref[...] = egA.astype(i32)
    ega2_ref[...] = egA2.astype(i32)
    egb1_ref[...] = egB.astype(i32)
    egb2_ref[...] = egB2.astype(i32)


def _routing(ef2, G):
    nrow, nlane = ef2.shape
    i32 = jnp.int32
    return pl.pallas_call(
        _routing_body,
        out_shape=[jax.ShapeDtypeStruct((nrow, nlane), i32)] +
                  [jax.ShapeDtypeStruct((1, G), i32)] * 10,
    )(ef2)


def _gmm_body(ig_r, eg_r, lo_r, hi_r, ff_r, par_r, ega1_r, ega2_r,
              egb1_r, egb2_r,
              xs_r, w1a_r, w1b_r, b1_r, w2a_r, w2b_r, b2_r, ys_r):
    g = pl.program_id(0)
    T = ys_r.shape[0]
    E = b1_r.shape[0]
    lo = lo_r[0, g]
    hi = hi_r[0, g]
    base = ig_r[0, g] * T

    def do(w1_r, w2_r):
        eg = eg_r[0, g]
        rows = jax.lax.broadcasted_iota(jnp.int32, (E, 1), 0)
        b1 = jnp.sum(jnp.where(rows == eg, b1_r[...], 0.0), 0, keepdims=True)
        b2 = jnp.sum(jnp.where(rows == eg, b2_r[...], 0.0), 0, keepdims=True)
        x = xs_r[...]
        h = jax.lax.dot_general(x, w1_r[0], (((1,), (1,)), ((), ())),
                                preferred_element_type=jnp.float32)
        h = jnp.maximum(h + b1, 0.0)
        y = jax.lax.dot_general(h, w2_r[0], (((1,), (1,)), ((), ())),
                                preferred_element_type=jnp.float32)
        y = y + b2
        full = (lo == base) & (hi == base + T)

        @pl.when(full)
        def _():
            ys_r[...] = y

        @pl.when(~full)
        def _():
            r = base + jax.lax.broadcasted_iota(jnp.int32, (T, 1), 0)
            ym = jnp.where((r >= lo) & (r < hi), y, 0.0)

            @pl.when(ff_r[0, g] == 1)
            def _():
                ys_r[...] = ym

            @pl.when(ff_r[0, g] == 0)
            def _():
                ys_r[...] += ym

    p = par_r[0, g]

    @pl.when((lo < hi) & (p == 0))
    def _():
        do(w1a_r, w2a_r)

    @pl.when((lo < hi) & (p == 1))
    def _():
        do(w1b_r, w2b_r)


def _grouped_ffn(xs, W1, b1, W2, b2, sched):
    N, H = xs.shape
    E = W1.shape[0]
    ig = sched[0]
    G = ig.shape[1]
    T = _T
    imap_x = lambda g, *s: (s[0][0, g], 0)
    imap_a1 = lambda g, *s: (s[6][0, g], 0, 0)
    imap_a2 = lambda g, *s: (s[7][0, g], 0, 0)
    imap_b1 = lambda g, *s: (s[8][0, g], 0, 0)
    imap_b2 = lambda g, *s: (s[9][0, g], 0, 0)
    imap_c = lambda g, *s: (0, 0)
    grid_spec = pltpu.PrefetchScalarGridSpec(
        num_scalar_prefetch=10,
        grid=(G,),
        in_specs=[
            pl.BlockSpec((T, H), imap_x),
            pl.BlockSpec((1, H, H), imap_a1),
            pl.BlockSpec((1, H, H), imap_b1),
            pl.BlockSpec((E, H), imap_c),
            pl.BlockSpec((1, H, H), imap_a2),
            pl.BlockSpec((1, H, H), imap_b2),
            pl.BlockSpec((E, H), imap_c),
        ],
        out_specs=pl.BlockSpec((T, H), imap_x),
    )
    return pl.pallas_call(
        _gmm_body,
        grid_spec=grid_spec,
        out_shape=jax.ShapeDtypeStruct((N, H), jnp.float32),
    )(*sched, xs, W1, W1, b1, W2, W2, b2)


def _dg16(v, idx):
    dnums = lax.GatherDimensionNumbers(
        offset_dims=(), collapsed_slice_dims=(0,), start_index_map=(0,))
    return lax.gather(v, idx[:, None], dnums, (1,),
                      mode=lax.GatherScatterMode.PROMISE_IN_BOUNDS)


def _sc_scatter(x2d, dest2, K):
    NT, H = x2d.shape
    NW, P = dest2.shape
    N = NW * P
    tpw = NT // NW

    @functools.partial(
        pl.kernel,
        mesh=plsc.VectorSubcoreMesh(core_axis_name="c", subcore_axis_name="s"),
        out_type=jax.ShapeDtypeStruct((N, H), jnp.float32),
        scratch_types=[pltpu.VMEM((P,), jnp.int32),
                       pltpu.VMEM((tpw,), jnp.int32),
                       pltpu.VMEM((tpw,), jnp.int32),
                       pltpu.VMEM((tpw, H), jnp.float32),
                       pltpu.SemaphoreType.DMA],
    )
    def k(x_hbm, dest_hbm, xs_hbm, dch_v, ev_v, od_v, rows_v, sem):
        wid = lax.axis_index("s") * 2 + lax.axis_index("c")
        dx = pltpu.async_copy(x_hbm.at[pl.ds(wid * tpw, tpw)], rows_v, sem)
        pltpu.sync_copy(dest_hbm.at[wid], dch_v)
        ip = lax.iota(jnp.int32, 16)
        half = (ip < 8)
        evi = (ip % 8) * 2
        for c in range(tpw // 16):
            c0 = dch_v[pl.ds(32 * c, 16)]
            c1 = dch_v[pl.ds(32 * c + 16, 16)]
            ev_v[pl.ds(16 * c, 16)] = jnp.where(
                half, _dg16(c0, evi), _dg16(c1, evi))
            od_v[pl.ds(16 * c, 16)] = jnp.where(
                half, _dg16(c0, evi + 1), _dg16(c1, evi + 1))
        dx.wait()
        d1 = pltpu.async_copy(rows_v, xs_hbm.at[ev_v], sem)
        d2 = pltpu.async_copy(rows_v, xs_hbm.at[od_v], sem)
        d1.wait()
        d2.wait()

    assert K == 2 and P == K * tpw
    return k(x2d, dest2)


def _sc_combine(ys, dest2, wflat, NT):
    N, H = ys.shape
    NW, P = dest2.shape
    tpw = NT // NW
    nq = 2
    hp = P // nq
    tph = tpw // nq

    @functools.partial(
        pl.kernel,
        mesh=plsc.VectorSubcoreMesh(core_axis_name="c", subcore_axis_name="s"),
        out_type=jax.ShapeDtypeStruct((NT, H), jnp.float32),
        scratch_types=[pltpu.VMEM((hp,), jnp.int32),
                       pltpu.VMEM((hp,), jnp.int32),
                       pltpu.VMEM((P,), jnp.float32),
                       pltpu.VMEM((hp, H), jnp.float32),
                       pltpu.VMEM((hp, H), jnp.float32),
                       pltpu.VMEM((tph, H), jnp.float32),
                       pltpu.SemaphoreType.DMA,
                       pltpu.SemaphoreType.DMA],
    )
    def k(ys_hbm, dest_hbm, w_hbm, out_hbm,
          idxA, idxB, w_v, bufA, bufB, obuf_v, semA, semB):
        wid = lax.axis_index("s") * 2 + lax.axis_index("c")
        pltpu.sync_copy(w_hbm.at[pl.ds(wid * P, P)], w_v)
        idxs = [idxA, idxB]
        bufs = [bufA, bufB]
        sems = [semA, semB]
        pltpu.sync_copy(dest_hbm.at[wid, pl.ds(0, hp)], idxA)
        dma = [pltpu.async_copy(ys_hbm.at[idxA], bufA, semA), None]
        for q in range(nq):
            cur = q % 2
            if q + 1 < nq:
                nxt = (q + 1) % 2
                pltpu.sync_copy(
                    dest_hbm.at[wid, pl.ds((q + 1) * hp, hp)], idxs[nxt])
                dma[nxt] = pltpu.async_copy(
                    ys_hbm.at[idxs[nxt]], bufs[nxt], sems[nxt])
            dma[cur].wait()
            buf_v = bufs[cur]

            @plsc.parallel_loop(0, tph, 1, unroll=4)
            def body(j):
                jj = j + q * tph
                b = jnp.minimum(2 * jj, P - 16)
                o = 2 * jj - b
                wv = w_v[pl.ds(b, 16)]
                z = jnp.zeros((16,), jnp.int32)
                s0 = _dg16(wv, z + o)
                s1 = _dg16(wv, z + o + 1)
                for c in range(H // 16):
                    s = pl.ds(c * 16, 16)
                    obuf_v[j, s] = s0 * buf_v[2 * j, s] + s1 * buf_v[2 * j + 1, s]
            pltpu.sync_copy(obuf_v, out_hbm.at[pl.ds(wid * tpw + q * tph, tph)])

    return k(ys, dest2, wflat)


def kernel(hidden_states, top_k_index, top_k_weights, W1, b1, W2, b2):
    B, S, H = hidden_states.shape
    E = W1.shape[0]
    NT = B * S
    K = top_k_index.shape[-1]
    N = NT * K
    NW = 32
    G = N // _T + E - 1

    x2d = hidden_states.reshape(NT, H)
    ef2 = top_k_index.astype(jnp.int32).reshape(N // 128, 128)
    wflat = top_k_weights.reshape(N)

    routed = _routing(ef2, G)
    dest2 = routed[0]
    sched = routed[1:]
    destw = dest2.reshape(NW, N // NW)

    xs = _sc_scatter(x2d, destw, K)
    ys = _grouped_ffn(xs, W1, b1, W2, b2, sched)
    out = _sc_combine(ys, destw, wflat, NT)
    return out.reshape(B, S, H)
